# baseline (speedup 1.0000x reference)
import jax
import jax.numpy as jnp
from jax import lax
from jax.experimental import pallas as pl
from jax.experimental.pallas import tpu as pltpu
from jax.experimental.pallas import tpu_sc as plsc

N = 10000
E = 160000
D = 256
H = 128
NC, NS, L = 2, 16, 16
EP = 163840
DUMMY = N
DEG_P = 10240
ACC_R = 10240
RB = 2000
GRID = N // RB

EB1 = EP // (NC * NS)
NB1 = EB1 // 128


def _k1_body(row_hbm, col_hbm, deg_out, didx_out, col0_out, coln_out,
             acc, rowb, colb, didx2, coln2, wval, zbuf, ssem):
    c = lax.axis_index("c")
    s = lax.axis_index("s")

    def zb(i, _):
        zbuf[pl.ds(i * L, L)] = jnp.zeros((L,), jnp.float32)
        return 0

    lax.fori_loop(0, 640 // L, zb, 0)
    pltpu.sync_copy(zbuf, acc.at[pl.ds(s * 640, 640)])
    plsc.subcore_barrier()

    i = c * NS + s
    base = i * (E // (NC * NS))
    pltpu.sync_copy(row_hbm.at[pl.ds(base, E // (NC * NS))],
                    rowb.at[pl.ds(0, E // (NC * NS))])
    pltpu.sync_copy(col_hbm.at[pl.ds(base, E // (NC * NS))],
                    colb.at[pl.ds(0, E // (NC * NS))])

    def mk(g, _):
        ii = lax.iota(jnp.int32, L)
        for l in range(128 // L):
            e0 = g * 128 + l * L
            sl = pl.ds(e0, L)
            sl2 = pl.ds(l * L, L)
            valid = (e0 + ii) < (E // (NC * NS))
            rv = rowb[sl]
            cv = jnp.where(valid, colb[sl], e0 + ii)
            wval[g, sl2] = jnp.where(valid & (rv != cv), jnp.float32(1.0),
                                     jnp.float32(0.0))
            dummy = DUMMY + lax.rem(g * 8 + l, 15) * L + ii
            didx2[g, sl2] = jnp.where(valid & (rv != cv), rv, dummy)
            colb[sl] = cv
            coln2[g, sl2] = cv + N
        return 0

    lax.fori_loop(0, NB1, mk, 0)
    pltpu.sync_copy(didx2, didx_out.at[i])
    pltpu.sync_copy(colb, col0_out.at[i])
    pltpu.sync_copy(coln2, coln_out.at[i])

    def sc(kk, _):
        for j in range(8):
            g = kk * 8 + j
            pltpu.async_copy(wval.at[g], acc.at[didx2.at[g]], ssem,
                             add=True)
        for j in range(8):
            g = kk * 8 + j
            pltpu.make_async_copy(wval.at[g], acc.at[didx2.at[g]],
                                  ssem).wait()
        return 0

    lax.fori_loop(0, NB1 // 8, sc, 0)
    plsc.subcore_barrier()
    pltpu.sync_copy(acc.at[pl.ds(s * 640, 640)],
                    deg_out.at[c, pl.ds(s * 640, 640)])


_k1 = pl.kernel(
    _k1_body,
    out_type=[
        jax.ShapeDtypeStruct((NC, DEG_P), jnp.float32),
        jax.ShapeDtypeStruct((NC * NS, NB1, 128), jnp.int32),
        jax.ShapeDtypeStruct((NC * NS, EB1), jnp.int32),
        jax.ShapeDtypeStruct((NC * NS, NB1, 128), jnp.int32),
    ],
    mesh=plsc.VectorSubcoreMesh(core_axis_name="c", subcore_axis_name="s"),
    scratch_types=[
        pltpu.VMEM_SHARED((DEG_P,), jnp.float32),
        pltpu.VMEM((EB1,), jnp.int32),
        pltpu.VMEM((EB1,), jnp.int32),
        pltpu.VMEM((NB1, 128), jnp.int32),
        pltpu.VMEM((NB1, 128), jnp.int32),
        pltpu.VMEM((NB1, 128), jnp.float32),
        pltpu.VMEM((640,), jnp.float32),
        pltpu.SemaphoreType.DMA,
    ],
)

EB3 = EP // NS
B3 = 128
NB3 = EB3 // B3
IR = 4


def _k3_body(col_hbm, coln_hbm, didx_hbm, z_hbm, inner_out,
             acc, gring, dring, fbuf,
             gsem0, gsem1, ssem0, ssem1, isem0, isem1, isem2, isem3):
    c = lax.axis_index("c")
    s = lax.axis_index("s")
    gsems = [gsem0, gsem1]
    ssems = [ssem0, ssem1]
    isems = [isem0, isem1, isem2, isem3]

    coff = c * N
    r0 = s * 640

    @pl.when(s != NS - 1)
    def _():
        pltpu.sync_copy(z_hbm.at[pl.ds(coff + r0, 640)],
                        acc.at[pl.ds(r0, 640)])

    @pl.when(s == NS - 1)
    def _():
        pltpu.sync_copy(z_hbm.at[pl.ds(coff + r0, 400)],
                        acc.at[pl.ds(r0, 400)])

    plsc.subcore_barrier()

    base = s * EB3

    def load_idx(g, islot):
        off = base + g * B3

        @pl.when(c == 0)
        def _():
            pltpu.async_copy(col_hbm.at[pl.ds(off, B3)], gring.at[islot],
                             isems[islot])

        @pl.when(c != 0)
        def _():
            pltpu.async_copy(coln_hbm.at[pl.ds(off, B3)], gring.at[islot],
                             isems[islot])

        pltpu.async_copy(didx_hbm.at[pl.ds(off, B3)], dring.at[islot],
                         isems[islot])

    def wait_idx(g, islot):
        off = base + g * B3

        @pl.when(c == 0)
        def _():
            pltpu.make_async_copy(col_hbm.at[pl.ds(off, B3)],
                                  gring.at[islot], isems[islot]).wait()

        @pl.when(c != 0)
        def _():
            pltpu.make_async_copy(coln_hbm.at[pl.ds(off, B3)],
                                  gring.at[islot], isems[islot]).wait()

        pltpu.make_async_copy(didx_hbm.at[pl.ds(off, B3)],
                              dring.at[islot], isems[islot]).wait()

    def gather(islot, slot):
        pltpu.async_copy(z_hbm.at[gring.at[islot]], fbuf.at[slot],
                         gsems[slot])

    def wait_gather(islot, slot):
        pltpu.make_async_copy(z_hbm.at[gring.at[islot]], fbuf.at[slot],
                              gsems[slot]).wait()

    def scatter(islot, slot):
        pltpu.async_copy(fbuf.at[slot], acc.at[dring.at[islot]],
                         ssems[slot], add=True)

    def wait_scatter(islot, slot):
        pltpu.make_async_copy(fbuf.at[slot], acc.at[dring.at[islot]],
                              ssems[slot]).wait()

    load_idx(0, 0)
    load_idx(1, 1)
    load_idx(2, 2)
    wait_idx(0, 0)
    gather(0, 0)
    load_idx(3, 3)
    wait_idx(1, 1)
    gather(1, 1)
    wait_gather(0, 0)
    scatter(0, 0)
    wait_scatter(0, 0)
    load_idx(4, 0)
    wait_idx(2, 2)
    gather(2, 0)
    wait_gather(1, 1)
    scatter(1, 1)
    wait_scatter(1, 1)
    load_idx(5, 1)
    wait_idx(3, 3)
    gather(3, 1)
    wait_gather(2, 0)
    scatter(2, 0)

    def body(k, _):
        g0 = k * 4
        for b in range(4):
            g = g0 + b
            sl = b % 2
            osl = (b + 1) % 2
            wait_scatter((b + 2) % 4, sl)

            @pl.when(g + 2 < NB3)
            def _():
                load_idx(g + 2, (b + 2) % 4)

            wait_idx(g, b)
            gather(b, sl)
            wait_gather((b + 3) % 4, osl)
            scatter((b + 3) % 4, osl)
        return 0

    lax.fori_loop(1, NB3 // 4, body, 0)
    wait_gather(3, 1)
    scatter(3, 1)
    wait_scatter(2, 0)
    wait_scatter(3, 1)
    plsc.subcore_barrier()

    def co(k, _):
        r = s * 640 + k * 64
        pltpu.async_copy(acc.at[pl.ds(r, 64)],
                         inner_out.at[c, pl.ds(r, 64)], isem0)
        return 0

    lax.fori_loop(0, 10, co, 0)

    def cow(k, _):
        r = s * 640 + k * 64
        pltpu.make_async_copy(acc.at[pl.ds(r, 64)],
                              inner_out.at[c, pl.ds(r, 64)], isem0).wait()
        return 0

    lax.fori_loop(0, 10, cow, 0)


_k3 = pl.kernel(
    _k3_body,
    out_type=jax.ShapeDtypeStruct((NC, ACC_R, H), jnp.float32),
    mesh=plsc.VectorSubcoreMesh(core_axis_name="c", subcore_axis_name="s"),
    scratch_types=[
        pltpu.VMEM_SHARED((ACC_R, H), jnp.float32),
        pltpu.VMEM((IR, B3), jnp.int32),
        pltpu.VMEM((IR, B3), jnp.int32),
        pltpu.VMEM((2, B3, H), jnp.float32),
        pltpu.SemaphoreType.DMA,
        pltpu.SemaphoreType.DMA,
        pltpu.SemaphoreType.DMA,
        pltpu.SemaphoreType.DMA,
        pltpu.SemaphoreType.DMA,
        pltpu.SemaphoreType.DMA,
        pltpu.SemaphoreType.DMA,
        pltpu.SemaphoreType.DMA,
    ],
)


def _k2_body(x_ref, d0_ref, d1_ref, dis_ref, z_ref):
    d = d0_ref[...] + d1_ref[...] + 1.0
    dis = lax.rsqrt(d)
    dis_ref[...] = dis
    z = dis * x_ref[...]
    z_ref[0] = z[:, :H]
    z_ref[1] = z[:, H:]


_k2 = pl.pallas_call(
    _k2_body,
    grid=(GRID,),
    in_specs=[
        pl.BlockSpec((RB, D), lambda j: (j, 0)),
        pl.BlockSpec((RB, 1), lambda j: (j, 0)),
        pl.BlockSpec((RB, 1), lambda j: (j, 0)),
    ],
    out_specs=[
        pl.BlockSpec((RB, 1), lambda j: (j, 0)),
        pl.BlockSpec((2, RB, H), lambda j: (0, j, 0)),
    ],
    out_shape=[
        jax.ShapeDtypeStruct((N, 1), jnp.float32),
        jax.ShapeDtypeStruct((2, N, H), jnp.float32),
    ],
)


def _k4_body(inner_ref, dis_ref, w_ref, b_ref, o_ref):
    dis = dis_ref[...]
    a0 = dis * inner_ref[0]
    a1 = dis * inner_ref[1]
    acc = lax.dot_general(a0, w_ref[:, :H], (((1,), (1,)), ((), ())),
                          preferred_element_type=jnp.float32)
    acc = acc + lax.dot_general(a1, w_ref[:, H:], (((1,), (1,)), ((), ())),
                                preferred_element_type=jnp.float32)
    o_ref[...] = jnp.maximum(acc + b_ref[...], 0.0)


_k4 = pl.pallas_call(
    _k4_body,
    grid=(GRID,),
    in_specs=[
        pl.BlockSpec((2, RB, H), lambda j: (0, j, 0)),
        pl.BlockSpec((RB, 1), lambda j: (j, 0)),
        pl.BlockSpec((D, D), lambda j: (0, 0)),
        pl.BlockSpec((1, D), lambda j: (0, 0)),
    ],
    out_specs=pl.BlockSpec((RB, D), lambda j: (j, 0)),
    out_shape=jax.ShapeDtypeStruct((N, D), jnp.float32),
)


def kernel(x, edge_index, W, b):
    ei = edge_index.astype(jnp.int32)

    degp, didx3, col03, coln3 = _k1(ei[0], ei[1])
    d0 = degp[0, :N].reshape(N, 1)
    d1 = degp[1, :N].reshape(N, 1)

    dis, z = _k2(x, d0, d1)
    zf = z.reshape(2 * N, H)

    inner = _k3(col03.reshape(EP), coln3.reshape(EP), didx3.reshape(EP), zf)

    return _k4(inner, dis, W, b.reshape(1, D))


# speedup vs baseline: 1.0125x; 1.0125x over previous
import jax
import jax.numpy as jnp
from jax import lax
from jax.experimental import pallas as pl
from jax.experimental.pallas import tpu as pltpu
from jax.experimental.pallas import tpu_sc as plsc

N = 10000
E = 160000
D = 256
H = 128
NC, NS, L = 2, 16, 16
EP = 163840
DUMMY = N
DEG_P = 10240
ACC_R = 10240
RB = 2000
GRID = N // RB

EB1 = EP // (NC * NS)
NB1 = EB1 // 128


def _k1_body(row_hbm, col_hbm, deg_out, didx_out, coln_out,
             acc, rowb, colb, wval, zbuf, ssem):
    c = lax.axis_index("c")
    s = lax.axis_index("s")

    def zb(i, _):
        zbuf[pl.ds(i * L, L)] = jnp.zeros((L,), jnp.float32)
        return 0

    lax.fori_loop(0, 640 // L, zb, 0)
    pltpu.sync_copy(zbuf, acc.at[pl.ds(s * 640, 640)])
    plsc.subcore_barrier()

    i = c * NS + s
    pltpu.sync_copy(row_hbm.at[i], rowb)
    pltpu.sync_copy(col_hbm.at[i], colb)

    def mk(g, _):
        ii = lax.iota(jnp.int32, L)
        for l in range(128 // L):
            sl = pl.ds(l * L, L)
            rv = rowb[g, sl]
            cv = colb[g, sl]
            wval[g, sl] = jnp.where(rv != cv, jnp.float32(1.0),
                                    jnp.float32(0.0))
            dummy = DUMMY + lax.rem(g * 8 + l, 15) * L + ii
            rowb[g, sl] = jnp.where(rv == cv, dummy, rv)
            colb[g, sl] = cv + N
        return 0

    lax.fori_loop(0, NB1, mk, 0)
    pltpu.sync_copy(rowb, didx_out.at[i])
    pltpu.sync_copy(colb, coln_out.at[i])

    def sc(kk, _):
        for j in range(8):
            g = kk * 8 + j
            pltpu.async_copy(wval.at[g], acc.at[rowb.at[g]], ssem, add=True)
        for j in range(8):
            g = kk * 8 + j
            pltpu.make_async_copy(wval.at[g], acc.at[rowb.at[g]],
                                  ssem).wait()
        return 0

    lax.fori_loop(0, NB1 // 8, sc, 0)
    plsc.subcore_barrier()
    pltpu.sync_copy(acc.at[pl.ds(s * 640, 640)],
                    deg_out.at[c, pl.ds(s * 640, 640)])


_k1 = pl.kernel(
    _k1_body,
    out_type=[
        jax.ShapeDtypeStruct((NC, DEG_P), jnp.float32),
        jax.ShapeDtypeStruct((NC * NS, NB1, 128), jnp.int32),
        jax.ShapeDtypeStruct((NC * NS, NB1, 128), jnp.int32),
    ],
    mesh=plsc.VectorSubcoreMesh(core_axis_name="c", subcore_axis_name="s"),
    scratch_types=[
        pltpu.VMEM_SHARED((DEG_P,), jnp.float32),
        pltpu.VMEM((NB1, 128), jnp.int32),
        pltpu.VMEM((NB1, 128), jnp.int32),
        pltpu.VMEM((NB1, 128), jnp.float32),
        pltpu.VMEM((640,), jnp.float32),
        pltpu.SemaphoreType.DMA,
    ],
)

EB3 = EP // NS
B3 = 128
NB3 = EB3 // B3
IR = 4


def _k3_body(col_hbm, coln_hbm, didx_hbm, z_hbm, inner_out,
             acc, gring, dring, fbuf,
             gsem0, gsem1, ssem0, ssem1, isem0, isem1, isem2, isem3):
    c = lax.axis_index("c")
    s = lax.axis_index("s")
    gsems = [gsem0, gsem1]
    ssems = [ssem0, ssem1]
    isems = [isem0, isem1, isem2, isem3]

    coff = c * N
    r0 = s * 640

    @pl.when(s != NS - 1)
    def _():
        pltpu.sync_copy(z_hbm.at[pl.ds(coff + r0, 640)],
                        acc.at[pl.ds(r0, 640)])

    @pl.when(s == NS - 1)
    def _():
        pltpu.sync_copy(z_hbm.at[pl.ds(coff + r0, 400)],
                        acc.at[pl.ds(r0, 400)])

    plsc.subcore_barrier()

    base = s * EB3

    def load_idx(g, islot):
        off = base + g * B3

        @pl.when(c == 0)
        def _():
            pltpu.async_copy(col_hbm.at[pl.ds(off, B3)], gring.at[islot],
                             isems[islot])

        @pl.when(c != 0)
        def _():
            pltpu.async_copy(coln_hbm.at[pl.ds(off, B3)], gring.at[islot],
                             isems[islot])

        pltpu.async_copy(didx_hbm.at[pl.ds(off, B3)], dring.at[islot],
                         isems[islot])

    def wait_idx(g, islot):
        off = base + g * B3

        @pl.when(c == 0)
        def _():
            pltpu.make_async_copy(col_hbm.at[pl.ds(off, B3)],
                                  gring.at[islot], isems[islot]).wait()

        @pl.when(c != 0)
        def _():
            pltpu.make_async_copy(coln_hbm.at[pl.ds(off, B3)],
                                  gring.at[islot], isems[islot]).wait()

        pltpu.make_async_copy(didx_hbm.at[pl.ds(off, B3)],
                              dring.at[islot], isems[islot]).wait()

    def gather(islot, slot):
        pltpu.async_copy(z_hbm.at[gring.at[islot]], fbuf.at[slot],
                         gsems[slot])

    def wait_gather(islot, slot):
        pltpu.make_async_copy(z_hbm.at[gring.at[islot]], fbuf.at[slot],
                              gsems[slot]).wait()

    def scatter(islot, slot):
        pltpu.async_copy(fbuf.at[slot], acc.at[dring.at[islot]],
                         ssems[slot], add=True)

    def wait_scatter(islot, slot):
        pltpu.make_async_copy(fbuf.at[slot], acc.at[dring.at[islot]],
                              ssems[slot]).wait()

    load_idx(0, 0)
    load_idx(1, 1)
    load_idx(2, 2)
    wait_idx(0, 0)
    gather(0, 0)
    load_idx(3, 3)
    wait_idx(1, 1)
    gather(1, 1)
    wait_gather(0, 0)
    scatter(0, 0)
    wait_scatter(0, 0)
    load_idx(4, 0)
    wait_idx(2, 2)
    gather(2, 0)
    wait_gather(1, 1)
    scatter(1, 1)
    wait_scatter(1, 1)
    load_idx(5, 1)
    wait_idx(3, 3)
    gather(3, 1)
    wait_gather(2, 0)
    scatter(2, 0)

    def body(k, _):
        g0 = k * 4
        for b in range(4):
            g = g0 + b
            sl = b % 2
            osl = (b + 1) % 2
            wait_scatter((b + 2) % 4, sl)

            @pl.when(g + 2 < NB3)
            def _():
                load_idx(g + 2, (b + 2) % 4)

            wait_idx(g, b)
            gather(b, sl)
            wait_gather((b + 3) % 4, osl)
            scatter((b + 3) % 4, osl)
        return 0

    lax.fori_loop(1, NB3 // 4, body, 0)
    wait_gather(3, 1)
    scatter(3, 1)
    wait_scatter(2, 0)
    wait_scatter(3, 1)
    plsc.subcore_barrier()

    def co(k, _):
        r = s * 640 + k * 64
        pltpu.async_copy(acc.at[pl.ds(r, 64)],
                         inner_out.at[c, pl.ds(r, 64)], isem0)
        return 0

    lax.fori_loop(0, 10, co, 0)

    def cow(k, _):
        r = s * 640 + k * 64
        pltpu.make_async_copy(acc.at[pl.ds(r, 64)],
                              inner_out.at[c, pl.ds(r, 64)], isem0).wait()
        return 0

    lax.fori_loop(0, 10, cow, 0)


_k3 = pl.kernel(
    _k3_body,
    out_type=jax.ShapeDtypeStruct((NC, ACC_R, H), jnp.float32),
    mesh=plsc.VectorSubcoreMesh(core_axis_name="c", subcore_axis_name="s"),
    scratch_types=[
        pltpu.VMEM_SHARED((ACC_R, H), jnp.float32),
        pltpu.VMEM((IR, B3), jnp.int32),
        pltpu.VMEM((IR, B3), jnp.int32),
        pltpu.VMEM((2, B3, H), jnp.float32),
        pltpu.SemaphoreType.DMA,
        pltpu.SemaphoreType.DMA,
        pltpu.SemaphoreType.DMA,
        pltpu.SemaphoreType.DMA,
        pltpu.SemaphoreType.DMA,
        pltpu.SemaphoreType.DMA,
        pltpu.SemaphoreType.DMA,
        pltpu.SemaphoreType.DMA,
    ],
)


def _k2_body(x_ref, d0_ref, d1_ref, dis_ref, z_ref):
    d = d0_ref[...] + d1_ref[...] + 1.0
    dis = lax.rsqrt(d)
    dis_ref[...] = dis
    z = dis * x_ref[...]
    z_ref[0] = z[:, :H]
    z_ref[1] = z[:, H:]


_k2 = pl.pallas_call(
    _k2_body,
    grid=(GRID,),
    in_specs=[
        pl.BlockSpec((RB, D), lambda j: (j, 0)),
        pl.BlockSpec((RB, 1), lambda j: (j, 0)),
        pl.BlockSpec((RB, 1), lambda j: (j, 0)),
    ],
    out_specs=[
        pl.BlockSpec((RB, 1), lambda j: (j, 0)),
        pl.BlockSpec((2, RB, H), lambda j: (0, j, 0)),
    ],
    out_shape=[
        jax.ShapeDtypeStruct((N, 1), jnp.float32),
        jax.ShapeDtypeStruct((2, N, H), jnp.float32),
    ],
)


def _k4_body(inner_ref, dis_ref, w_ref, b_ref, o_ref):
    dis = dis_ref[...]
    a0 = dis * inner_ref[0]
    a1 = dis * inner_ref[1]
    acc = lax.dot_general(a0, w_ref[:, :H], (((1,), (1,)), ((), ())),
                          preferred_element_type=jnp.float32)
    acc = acc + lax.dot_general(a1, w_ref[:, H:], (((1,), (1,)), ((), ())),
                                preferred_element_type=jnp.float32)
    o_ref[...] = jnp.maximum(acc + b_ref[...], 0.0)


_k4 = pl.pallas_call(
    _k4_body,
    grid=(GRID,),
    in_specs=[
        pl.BlockSpec((2, RB, H), lambda j: (0, j, 0)),
        pl.BlockSpec((RB, 1), lambda j: (j, 0)),
        pl.BlockSpec((D, D), lambda j: (0, 0)),
        pl.BlockSpec((1, D), lambda j: (0, 0)),
    ],
    out_specs=pl.BlockSpec((RB, D), lambda j: (j, 0)),
    out_shape=jax.ShapeDtypeStruct((N, D), jnp.float32),
)


def kernel(x, edge_index, W, b):
    row = edge_index[0].astype(jnp.int32)
    col = edge_index[1].astype(jnp.int32)
    ar = jnp.arange(EP - E, dtype=jnp.int32)
    row_p = jnp.concatenate([row, DUMMY + ar % 240])
    col_p = jnp.concatenate([col, (ar * 41) % N])
    row3 = row_p.reshape(NC * NS, NB1, 128)
    col3 = col_p.reshape(NC * NS, NB1, 128)

    degp, didx3, coln3 = _k1(row3, col3)
    d0 = degp[0, :N].reshape(N, 1)
    d1 = degp[1, :N].reshape(N, 1)

    dis, z = _k2(x, d0, d1)
    zf = z.reshape(2 * N, H)

    inner = _k3(col_p, coln3.reshape(EP), didx3.reshape(EP), zf)

    return _k4(inner, dis, W, b.reshape(1, D))
